# baseline (speedup 1.0000x reference)
import functools

import jax
import jax.numpy as jnp
from jax import lax
from jax.experimental import pallas as pl
from jax.experimental.pallas import tpu as pltpu
from jax.experimental.pallas import tpu_sc as plsc

N = 10000
E = 160000
D = 256
H = 128
NPAD = 10112
ER = 1280
EPAD = ER * 128
NC = 2
NS = 16
NW = NC * NS
BR = 128
CHUNK = 64

def _mesh():
  return plsc.VectorSubcoreMesh(
      core_axis_name="c", subcore_axis_name="s", num_cores=NC, num_subcores=NS)


def _sc_deg(dst2, ones, zeros16):
  rows_per_tile = ER // NW
  nslice = NPAD // NS

  @functools.partial(
      pl.kernel,
      out_type=jax.ShapeDtypeStruct((NC, NPAD, 16), jnp.float32),
      mesh=_mesh(),
      scratch_types=[
          pltpu.VMEM((rows_per_tile, 128), jnp.int32),
          pltpu.VMEM((128, 16), jnp.float32),
          pltpu.VMEM_SHARED((NPAD, 16), jnp.float32),
      ],
  )
  def k(dst_hbm, ones_hbm, z_hbm, deg_hbm, dst_v, ones_v, deg_sh):
    cid = lax.axis_index("c")
    sid = lax.axis_index("s")
    wid = sid * NC + cid
    pltpu.sync_copy(dst_hbm.at[pl.ds(wid * rows_per_tile, rows_per_tile)],
                    dst_v)
    pltpu.sync_copy(ones_hbm, ones_v)
    pltpu.sync_copy(z_hbm, deg_sh.at[pl.ds(sid * nslice, nslice)])
    plsc.subcore_barrier()

    def body(j, carry):
      pltpu.sync_copy(ones_v, deg_sh.at[dst_v.at[j]], add=True)
      return carry

    lax.fori_loop(0, rows_per_tile, body, 0)
    plsc.subcore_barrier()
    sl = pl.ds(sid * nslice, nslice)
    pltpu.sync_copy(deg_sh.at[sl], deg_hbm.at[cid, sl])

  return k(dst2, ones, zeros16)


def _tc_prep(x_pad, W_conv, b_conv, deg_part):
  grid = NPAD // BR

  def body(x_ref, w_ref, b_ref, d_ref, g_ref):
    i = pl.program_id(0)
    deg = d_ref[0, :, 0:1] + d_ref[1, :, 0:1] + 1.0
    dinv = lax.rsqrt(deg)
    h = jnp.dot(x_ref[...], w_ref[...],
                preferred_element_type=jnp.float32) + b_ref[...][None, :]
    rows = i * BR + lax.broadcasted_iota(jnp.int32, (BR, 1), 0)
    g = jnp.where(rows < N, h * dinv, 0.0)
    g_ref[0] = g[:, :H]
    g_ref[1] = g[:, H:]

  return pl.pallas_call(
      body,
      grid=(grid,),
      in_specs=[
          pl.BlockSpec((BR, D), lambda i: (i, 0)),
          pl.BlockSpec((D, D), lambda i: (0, 0)),
          pl.BlockSpec((D,), lambda i: (0,)),
          pl.BlockSpec((NC, BR, 16), lambda i: (0, i, 0)),
      ],
      out_specs=pl.BlockSpec((NC, BR, H), lambda i: (0, i, 0)),
      out_shape=jax.ShapeDtypeStruct((NC, NPAD, H), jnp.float32),
  )(x_pad, W_conv, b_conv, deg_part)


def _sc_edge(src_pk, dst2, g_flat, zeros_ns):
  erows = EPAD // CHUNK
  rows_per_tile = erows // NS
  nslice = NPAD // NS
  nph = 4
  rows_per_ph = rows_per_tile // nph
  NB = 4

  @functools.partial(
      pl.kernel,
      out_type=jax.ShapeDtypeStruct((NC, NPAD, H), jnp.float32),
      mesh=_mesh(),
      scratch_types=[
          pltpu.VMEM((rows_per_ph, CHUNK), jnp.int32),
          pltpu.VMEM((rows_per_ph, CHUNK), jnp.int32),
      ] + [pltpu.VMEM((CHUNK, H), jnp.float32)] * NB
        + [pltpu.SemaphoreType.DMA] * (2 * NB)
        + [pltpu.VMEM_SHARED((NPAD, H), jnp.float32)],
  )
  def k(src_hbm, dst_hbm, g_hbm, z_hbm, s_hbm, src_v, dst_v,
        r0, r1, r2, r3, sg0, sg1, sg2, sg3, ss0, ss1, ss2, ss3, s_sh):
    rows = [r0, r1, r2, r3]
    sg = [sg0, sg1, sg2, sg3]
    ss = [ss0, ss1, ss2, ss3]
    cid = lax.axis_index("c")
    sid = lax.axis_index("s")
    pltpu.sync_copy(z_hbm, s_sh.at[pl.ds(sid * nslice, nslice)])
    plsc.subcore_barrier()

    def g_start(t, b):
      pltpu.async_copy(g_hbm.at[src_v.at[t]], rows[b], sg[b])

    def g_wait(t, b):
      pltpu.make_async_copy(g_hbm.at[src_v.at[t]], rows[b], sg[b]).wait()

    def s_start(t, b):
      pltpu.async_copy(rows[b], s_sh.at[dst_v.at[t]], ss[b], add=True)

    def s_wait(t, b):
      pltpu.make_async_copy(rows[b], s_sh.at[dst_v.at[t]], ss[b]).wait()

    for p in range(nph):
      base = sid * rows_per_tile + p * rows_per_ph
      pltpu.sync_copy(src_hbm.at[cid, pl.ds(base, rows_per_ph)], src_v)
      pltpu.sync_copy(dst_hbm.at[pl.ds(base, rows_per_ph)], dst_v)
      g_start(0, 0)
      g_start(1, 1)
      for t in range(4):
        if t >= 2:
          s_wait(t - 2, (t + 2) % NB)
        g_start(t + 2, (t + 2) % NB)
        g_wait(t, t % NB)
        s_start(t, t % NB)

      def body(jo, carry):
        for b in range(NB):
          t = jo * NB + b
          s_wait(t - 2, (b + 2) % NB)
          g_start(t + 2, (b + 2) % NB)
          g_wait(t, b)
          s_start(t, b)
        return carry

      lax.fori_loop(1, rows_per_ph // NB - 1, body, 0)
      for t in range(rows_per_ph - 4, rows_per_ph):
        b = t % NB
        s_wait(t - 2, (b + 2) % NB)
        if t + 2 < rows_per_ph:
          g_start(t + 2, (b + 2) % NB)
        g_wait(t, b)
        s_start(t, b)
      s_wait(rows_per_ph - 2, (rows_per_ph - 2) % NB)
      s_wait(rows_per_ph - 1, (rows_per_ph - 1) % NB)

    plsc.subcore_barrier()
    sl = pl.ds(sid * nslice, nslice)
    pltpu.sync_copy(s_sh.at[sl], s_hbm.at[cid, sl])

  return k(src_pk, dst2, g_flat, zeros_ns)


def _tc_final(S_pk, g_pk, deg_part, W1, b1, W2, b2):
  grid = NPAD // BR

  def body(s_ref, g_ref, d_ref, w1_ref, b1_ref, w2_ref, b2_ref,
           emb_ref, log_ref):
    deg = d_ref[0, :, 0:1] + d_ref[1, :, 0:1] + 1.0
    dinv = lax.rsqrt(deg)
    S = jnp.concatenate([s_ref[0], s_ref[1]], axis=1)
    g = jnp.concatenate([g_ref[0], g_ref[1]], axis=1)
    agg = (S + g) * dinv
    emb_ref[...] = agg
    X = jnp.maximum(agg, 0.0)
    X = jnp.dot(X, w1_ref[...],
                preferred_element_type=jnp.float32) + b1_ref[...][None, :]
    X = jnp.dot(X, w2_ref[...],
                preferred_element_type=jnp.float32) + b2_ref[...][None, :]
    m = jnp.max(X, axis=1, keepdims=True)
    lse = jnp.log(jnp.sum(jnp.exp(X - m), axis=1, keepdims=True)) + m
    log_ref[...] = X - lse

  return pl.pallas_call(
      body,
      grid=(grid,),
      in_specs=[
          pl.BlockSpec((NC, BR, H), lambda i: (0, i, 0)),
          pl.BlockSpec((NC, BR, H), lambda i: (0, i, 0)),
          pl.BlockSpec((NC, BR, 16), lambda i: (0, i, 0)),
          pl.BlockSpec((D, D), lambda i: (0, 0)),
          pl.BlockSpec((D,), lambda i: (0,)),
          pl.BlockSpec((D, D), lambda i: (0, 0)),
          pl.BlockSpec((D,), lambda i: (0,)),
      ],
      out_specs=[
          pl.BlockSpec((BR, D), lambda i: (i, 0)),
          pl.BlockSpec((BR, D), lambda i: (i, 0)),
      ],
      out_shape=[
          jax.ShapeDtypeStruct((N, D), jnp.float32),
          jax.ShapeDtypeStruct((N, D), jnp.float32),
      ],
  )(S_pk, g_pk, deg_part, W1, b1, W2, b2)


def kernel(x, edge_index, W_conv, b_conv, W1, b1, W2, b2):
  src = edge_index[0]
  dst = edge_index[1]
  pad = jnp.full((EPAD - E,), N, dtype=jnp.int32)
  srcp = jnp.concatenate([src, pad]).reshape(EPAD // CHUNK, CHUNK)
  dstp = jnp.concatenate([dst, pad]).reshape(EPAD // CHUNK, CHUNK)
  dstp128 = dstp.reshape(ER, 128)
  src_pk = jnp.stack([srcp, srcp + NPAD])

  ones16 = jnp.ones((128, 16), jnp.float32)
  zeros16 = jnp.zeros((NPAD // NS, 16), jnp.float32)
  zeros_ns = jnp.zeros((NPAD // NS, H), jnp.float32)

  deg_part = _sc_deg(dstp128, ones16, zeros16)
  g_pk = _tc_prep(x, W_conv, b_conv, deg_part)
  S_pk = _sc_edge(src_pk, dstp, g_pk.reshape(NC * NPAD, H), zeros_ns)
  emb, logits = _tc_final(S_pk, g_pk, deg_part, W1, b1, W2, b2)
  return emb, logits


# speedup vs baseline: 1.0000x; 1.0000x over previous
import functools

import jax
import jax.numpy as jnp
from jax import lax
from jax.experimental import pallas as pl
from jax.experimental.pallas import tpu as pltpu
from jax.experimental.pallas import tpu_sc as plsc

N = 10000
E = 160000
D = 256
H = 128
NPAD = 10112
ER = 1280
EPAD = ER * 128
NC = 2
NS = 16
NW = NC * NS
BR = 128
CHUNK = 128

def _mesh():
  return plsc.VectorSubcoreMesh(
      core_axis_name="c", subcore_axis_name="s", num_cores=NC, num_subcores=NS)


def _sc_deg(dst2, ones, zeros16):
  rows_per_tile = ER // NW
  nslice = NPAD // NS

  @functools.partial(
      pl.kernel,
      out_type=jax.ShapeDtypeStruct((NC, NPAD, 16), jnp.float32),
      mesh=_mesh(),
      scratch_types=[
          pltpu.VMEM((rows_per_tile, 128), jnp.int32),
          pltpu.VMEM((128, 16), jnp.float32),
          pltpu.VMEM_SHARED((NPAD, 16), jnp.float32),
      ],
  )
  def k(dst_hbm, ones_hbm, z_hbm, deg_hbm, dst_v, ones_v, deg_sh):
    cid = lax.axis_index("c")
    sid = lax.axis_index("s")
    wid = sid * NC + cid
    pltpu.sync_copy(dst_hbm.at[pl.ds(wid * rows_per_tile, rows_per_tile)],
                    dst_v)
    pltpu.sync_copy(ones_hbm, ones_v)
    pltpu.sync_copy(z_hbm, deg_sh.at[pl.ds(sid * nslice, nslice)])
    plsc.subcore_barrier()

    def body(j, carry):
      pltpu.sync_copy(ones_v, deg_sh.at[dst_v.at[j]], add=True)
      return carry

    lax.fori_loop(0, rows_per_tile, body, 0)
    plsc.subcore_barrier()
    sl = pl.ds(sid * nslice, nslice)
    pltpu.sync_copy(deg_sh.at[sl], deg_hbm.at[cid, sl])

  return k(dst2, ones, zeros16)


def _tc_prep(x_pad, W_conv, b_conv, deg_part):
  grid = NPAD // BR

  def body(x_ref, w_ref, b_ref, d_ref, g_ref):
    i = pl.program_id(0)
    deg = d_ref[0, :, 0:1] + d_ref[1, :, 0:1] + 1.0
    dinv = lax.rsqrt(deg)
    h = jnp.dot(x_ref[...], w_ref[...],
                preferred_element_type=jnp.float32) + b_ref[...][None, :]
    rows = i * BR + lax.broadcasted_iota(jnp.int32, (BR, 1), 0)
    g = jnp.where(rows < N, h * dinv, 0.0)
    g_ref[0] = g[:, :H]
    g_ref[1] = g[:, H:]

  return pl.pallas_call(
      body,
      grid=(grid,),
      in_specs=[
          pl.BlockSpec((BR, D), lambda i: (i, 0)),
          pl.BlockSpec((D, D), lambda i: (0, 0)),
          pl.BlockSpec((D,), lambda i: (0,)),
          pl.BlockSpec((NC, BR, 16), lambda i: (0, i, 0)),
      ],
      out_specs=pl.BlockSpec((NC, BR, H), lambda i: (0, i, 0)),
      out_shape=jax.ShapeDtypeStruct((NC, NPAD, H), jnp.float32),
  )(x_pad, W_conv, b_conv, deg_part)


def _sc_edge(src_pk, dst2, g_flat, zeros_ns):
  rows_per_tile = ER // NS
  nslice = NPAD // NS
  nph = 2
  rows_per_ph = rows_per_tile // nph

  @functools.partial(
      pl.kernel,
      out_type=jax.ShapeDtypeStruct((NC, NPAD, H), jnp.float32),
      mesh=_mesh(),
      scratch_types=[
          pltpu.VMEM((rows_per_ph, 128), jnp.int32),
          pltpu.VMEM((rows_per_ph, 128), jnp.int32),
          pltpu.VMEM((CHUNK, H), jnp.float32),
          pltpu.VMEM((CHUNK, H), jnp.float32),
          pltpu.SemaphoreType.DMA,
          pltpu.SemaphoreType.DMA,
          pltpu.VMEM_SHARED((NPAD, H), jnp.float32),
      ],
  )
  def k(src_hbm, dst_hbm, g_hbm, z_hbm, s_hbm,
        src_v, dst_v, rows0, rows1, sem0, sem1, s_sh):
    cid = lax.axis_index("c")
    sid = lax.axis_index("s")
    pltpu.sync_copy(z_hbm, s_sh.at[pl.ds(sid * nslice, nslice)])
    plsc.subcore_barrier()

    for p in range(nph):
      base = sid * rows_per_tile + p * rows_per_ph
      pltpu.sync_copy(src_hbm.at[cid, pl.ds(base, rows_per_ph)], src_v)
      pltpu.sync_copy(dst_hbm.at[pl.ds(base, rows_per_ph)], dst_v)
      pltpu.async_copy(g_hbm.at[src_v.at[0]], rows0, sem0)
      pltpu.async_copy(g_hbm.at[src_v.at[1]], rows1, sem1)

      def body(jo, carry):
        j0 = jo * 2
        pltpu.make_async_copy(g_hbm.at[src_v.at[j0]], rows0, sem0).wait()
        pltpu.sync_copy(rows0, s_sh.at[dst_v.at[j0]], add=True)
        pltpu.async_copy(g_hbm.at[src_v.at[j0 + 2]], rows0, sem0)
        pltpu.make_async_copy(g_hbm.at[src_v.at[j0 + 1]], rows1, sem1).wait()
        pltpu.sync_copy(rows1, s_sh.at[dst_v.at[j0 + 1]], add=True)
        pltpu.async_copy(g_hbm.at[src_v.at[j0 + 3]], rows1, sem1)
        return carry

      last = rows_per_ph - 2
      lax.fori_loop(0, (rows_per_ph - 2) // 2, body, 0)
      pltpu.make_async_copy(g_hbm.at[src_v.at[last]], rows0, sem0).wait()
      pltpu.sync_copy(rows0, s_sh.at[dst_v.at[last]], add=True)
      pltpu.make_async_copy(g_hbm.at[src_v.at[last + 1]], rows1, sem1).wait()
      pltpu.sync_copy(rows1, s_sh.at[dst_v.at[last + 1]], add=True)

    plsc.subcore_barrier()
    sl = pl.ds(sid * nslice, nslice)
    pltpu.sync_copy(s_sh.at[sl], s_hbm.at[cid, sl])

  return k(src_pk, dst2, g_flat, zeros_ns)


def _tc_final(S_pk, g_pk, deg_part, W1, b1, W2, b2):
  grid = NPAD // BR

  def body(s_ref, g_ref, d_ref, w1_ref, b1_ref, w2_ref, b2_ref,
           emb_ref, log_ref):
    deg = d_ref[0, :, 0:1] + d_ref[1, :, 0:1] + 1.0
    dinv = lax.rsqrt(deg)
    S = jnp.concatenate([s_ref[0], s_ref[1]], axis=1)
    g = jnp.concatenate([g_ref[0], g_ref[1]], axis=1)
    agg = (S + g) * dinv
    emb_ref[...] = agg
    X = jnp.maximum(agg, 0.0)
    X = jnp.dot(X, w1_ref[...],
                preferred_element_type=jnp.float32) + b1_ref[...][None, :]
    X = jnp.dot(X, w2_ref[...],
                preferred_element_type=jnp.float32) + b2_ref[...][None, :]
    m = jnp.max(X, axis=1, keepdims=True)
    lse = jnp.log(jnp.sum(jnp.exp(X - m), axis=1, keepdims=True)) + m
    log_ref[...] = X - lse

  return pl.pallas_call(
      body,
      grid=(grid,),
      in_specs=[
          pl.BlockSpec((NC, BR, H), lambda i: (0, i, 0)),
          pl.BlockSpec((NC, BR, H), lambda i: (0, i, 0)),
          pl.BlockSpec((NC, BR, 16), lambda i: (0, i, 0)),
          pl.BlockSpec((D, D), lambda i: (0, 0)),
          pl.BlockSpec((D,), lambda i: (0,)),
          pl.BlockSpec((D, D), lambda i: (0, 0)),
          pl.BlockSpec((D,), lambda i: (0,)),
      ],
      out_specs=[
          pl.BlockSpec((BR, D), lambda i: (i, 0)),
          pl.BlockSpec((BR, D), lambda i: (i, 0)),
      ],
      out_shape=[
          jax.ShapeDtypeStruct((N, D), jnp.float32),
          jax.ShapeDtypeStruct((N, D), jnp.float32),
      ],
  )(S_pk, g_pk, deg_part, W1, b1, W2, b2)


def kernel(x, edge_index, W_conv, b_conv, W1, b1, W2, b2):
  src = edge_index[0]
  dst = edge_index[1]
  pad = jnp.full((EPAD - E,), N, dtype=jnp.int32)
  srcp = jnp.concatenate([src, pad]).reshape(ER, 128)
  dstp = jnp.concatenate([dst, pad]).reshape(ER, 128)
  src_pk = jnp.stack([srcp, srcp + NPAD])

  ones16 = jnp.ones((128, 16), jnp.float32)
  zeros16 = jnp.zeros((NPAD // NS, 16), jnp.float32)
  zeros_ns = jnp.zeros((NPAD // NS, H), jnp.float32)

  deg_part = _sc_deg(dstp, ones16, zeros16)
  g_pk = _tc_prep(x, W_conv, b_conv, deg_part)
  S_pk = _sc_edge(src_pk, dstp, g_pk.reshape(NC * NPAD, H), zeros_ns)
  emb, logits = _tc_final(S_pk, g_pk, deg_part, W1, b1, W2, b2)
  return emb, logits


# speedup vs baseline: 1.2107x; 1.2106x over previous
import functools

import jax
import jax.numpy as jnp
from jax import lax
from jax.experimental import pallas as pl
from jax.experimental.pallas import tpu as pltpu
from jax.experimental.pallas import tpu_sc as plsc

N = 10000
E = 160000
D = 256
H = 128
NPAD = 10112
ER = 1280
EPAD = ER * 128
NC = 2
NS = 16
NW = NC * NS
BR = 1264
CHUNK = 128

def _mesh():
  return plsc.VectorSubcoreMesh(
      core_axis_name="c", subcore_axis_name="s", num_cores=NC, num_subcores=NS)


def _sc_deg(dst2, ones, zeros16):
  rows_per_tile = ER // NW
  nslice = NPAD // NS

  @functools.partial(
      pl.kernel,
      out_type=jax.ShapeDtypeStruct((NC, NPAD, 16), jnp.float32),
      mesh=_mesh(),
      scratch_types=[
          pltpu.VMEM((rows_per_tile, 128), jnp.int32),
          pltpu.VMEM((128, 16), jnp.float32),
          pltpu.VMEM_SHARED((NPAD, 16), jnp.float32),
      ],
  )
  def k(dst_hbm, ones_hbm, z_hbm, deg_hbm, dst_v, ones_v, deg_sh):
    cid = lax.axis_index("c")
    sid = lax.axis_index("s")
    wid = sid * NC + cid
    pltpu.sync_copy(dst_hbm.at[pl.ds(wid * rows_per_tile, rows_per_tile)],
                    dst_v)
    pltpu.sync_copy(ones_hbm, ones_v)
    pltpu.sync_copy(z_hbm, deg_sh.at[pl.ds(sid * nslice, nslice)])
    plsc.subcore_barrier()

    def body(j, carry):
      pltpu.sync_copy(ones_v, deg_sh.at[dst_v.at[j]], add=True)
      return carry

    lax.fori_loop(0, rows_per_tile, body, 0)
    plsc.subcore_barrier()
    sl = pl.ds(sid * nslice, nslice)
    pltpu.sync_copy(deg_sh.at[sl], deg_hbm.at[cid, sl])

  return k(dst2, ones, zeros16)


def _tc_prep(x_pad, W_conv, b_conv, deg_part):
  grid = NPAD // BR

  def body(x_ref, w_ref, b_ref, d_ref, g_ref):
    i = pl.program_id(0)
    deg = d_ref[0, :, 0:1] + d_ref[1, :, 0:1] + 1.0
    dinv = lax.rsqrt(deg)
    h = jnp.dot(x_ref[...], w_ref[...],
                preferred_element_type=jnp.float32) + b_ref[...][None, :]
    rows = i * BR + lax.broadcasted_iota(jnp.int32, (BR, 1), 0)
    g = jnp.where(rows < N, h * dinv, 0.0)
    g_ref[0] = g[:, :H]
    g_ref[1] = g[:, H:]

  return pl.pallas_call(
      body,
      grid=(grid,),
      in_specs=[
          pl.BlockSpec((BR, D), lambda i: (i, 0)),
          pl.BlockSpec((D, D), lambda i: (0, 0)),
          pl.BlockSpec((D,), lambda i: (0,)),
          pl.BlockSpec((NC, BR, 16), lambda i: (0, i, 0)),
      ],
      out_specs=pl.BlockSpec((NC, BR, H), lambda i: (0, i, 0)),
      out_shape=jax.ShapeDtypeStruct((NC, NPAD, H), jnp.float32),
  )(x_pad, W_conv, b_conv, deg_part)


def _sc_edge(src_pk, dst2, g_flat, zeros_ns):
  rows_per_tile = ER // NS
  nslice = NPAD // NS
  nph = 2
  rows_per_ph = rows_per_tile // nph

  @functools.partial(
      pl.kernel,
      out_type=jax.ShapeDtypeStruct((NC, NPAD, H), jnp.float32),
      mesh=_mesh(),
      scratch_types=[
          pltpu.VMEM((rows_per_ph, 128), jnp.int32),
          pltpu.VMEM((rows_per_ph, 128), jnp.int32),
          pltpu.VMEM((CHUNK, H), jnp.float32),
          pltpu.VMEM((CHUNK, H), jnp.float32),
          pltpu.SemaphoreType.DMA,
          pltpu.SemaphoreType.DMA,
          pltpu.VMEM_SHARED((NPAD, H), jnp.float32),
      ],
  )
  def k(src_hbm, dst_hbm, g_hbm, z_hbm, s_hbm,
        src_v, dst_v, rows0, rows1, sem0, sem1, s_sh):
    cid = lax.axis_index("c")
    sid = lax.axis_index("s")
    pltpu.sync_copy(z_hbm, s_sh.at[pl.ds(sid * nslice, nslice)])
    plsc.subcore_barrier()

    for p in range(nph):
      base = sid * rows_per_tile + p * rows_per_ph
      pltpu.sync_copy(src_hbm.at[cid, pl.ds(base, rows_per_ph)], src_v)
      pltpu.sync_copy(dst_hbm.at[pl.ds(base, rows_per_ph)], dst_v)
      pltpu.async_copy(g_hbm.at[src_v.at[0]], rows0, sem0)
      pltpu.async_copy(g_hbm.at[src_v.at[1]], rows1, sem1)

      def body(jo, carry):
        j0 = jo * 2
        pltpu.make_async_copy(g_hbm.at[src_v.at[j0]], rows0, sem0).wait()
        pltpu.sync_copy(rows0, s_sh.at[dst_v.at[j0]], add=True)
        pltpu.async_copy(g_hbm.at[src_v.at[j0 + 2]], rows0, sem0)
        pltpu.make_async_copy(g_hbm.at[src_v.at[j0 + 1]], rows1, sem1).wait()
        pltpu.sync_copy(rows1, s_sh.at[dst_v.at[j0 + 1]], add=True)
        pltpu.async_copy(g_hbm.at[src_v.at[j0 + 3]], rows1, sem1)
        return carry

      last = rows_per_ph - 2
      lax.fori_loop(0, (rows_per_ph - 2) // 2, body, 0)
      pltpu.make_async_copy(g_hbm.at[src_v.at[last]], rows0, sem0).wait()
      pltpu.sync_copy(rows0, s_sh.at[dst_v.at[last]], add=True)
      pltpu.make_async_copy(g_hbm.at[src_v.at[last + 1]], rows1, sem1).wait()
      pltpu.sync_copy(rows1, s_sh.at[dst_v.at[last + 1]], add=True)

    plsc.subcore_barrier()
    sl = pl.ds(sid * nslice, nslice)
    pltpu.sync_copy(s_sh.at[sl], s_hbm.at[cid, sl])

  return k(src_pk, dst2, g_flat, zeros_ns)


def _tc_final(S_pk, g_pk, deg_part, W1, b1, W2, b2):
  grid = NPAD // BR

  def body(s_ref, g_ref, d_ref, w1_ref, b1_ref, w2_ref, b2_ref,
           emb_ref, log_ref):
    deg = d_ref[0, :, 0:1] + d_ref[1, :, 0:1] + 1.0
    dinv = lax.rsqrt(deg)
    S = jnp.concatenate([s_ref[0], s_ref[1]], axis=1)
    g = jnp.concatenate([g_ref[0], g_ref[1]], axis=1)
    agg = (S + g) * dinv
    emb_ref[...] = agg
    X = jnp.maximum(agg, 0.0)
    X = jnp.dot(X, w1_ref[...],
                preferred_element_type=jnp.float32) + b1_ref[...][None, :]
    X = jnp.dot(X, w2_ref[...],
                preferred_element_type=jnp.float32) + b2_ref[...][None, :]
    m = jnp.max(X, axis=1, keepdims=True)
    lse = jnp.log(jnp.sum(jnp.exp(X - m), axis=1, keepdims=True)) + m
    log_ref[...] = X - lse

  return pl.pallas_call(
      body,
      grid=(grid,),
      in_specs=[
          pl.BlockSpec((NC, BR, H), lambda i: (0, i, 0)),
          pl.BlockSpec((NC, BR, H), lambda i: (0, i, 0)),
          pl.BlockSpec((NC, BR, 16), lambda i: (0, i, 0)),
          pl.BlockSpec((D, D), lambda i: (0, 0)),
          pl.BlockSpec((D,), lambda i: (0,)),
          pl.BlockSpec((D, D), lambda i: (0, 0)),
          pl.BlockSpec((D,), lambda i: (0,)),
      ],
      out_specs=[
          pl.BlockSpec((BR, D), lambda i: (i, 0)),
          pl.BlockSpec((BR, D), lambda i: (i, 0)),
      ],
      out_shape=[
          jax.ShapeDtypeStruct((NPAD, D), jnp.float32),
          jax.ShapeDtypeStruct((NPAD, D), jnp.float32),
      ],
  )(S_pk, g_pk, deg_part, W1, b1, W2, b2)


def kernel(x, edge_index, W_conv, b_conv, W1, b1, W2, b2):
  src = edge_index[0]
  dst = edge_index[1]
  pad = jnp.full((EPAD - E,), N, dtype=jnp.int32)
  srcp = jnp.concatenate([src, pad]).reshape(ER, 128)
  dstp = jnp.concatenate([dst, pad]).reshape(ER, 128)
  src_pk = jnp.stack([srcp, srcp + NPAD])
  x_pad = jnp.pad(x, ((0, NPAD - N), (0, 0)))

  ones16 = jnp.ones((128, 16), jnp.float32)
  zeros16 = jnp.zeros((NPAD // NS, 16), jnp.float32)
  zeros_ns = jnp.zeros((NPAD // NS, H), jnp.float32)

  deg_part = _sc_deg(dstp, ones16, zeros16)
  g_pk = _tc_prep(x_pad, W_conv, b_conv, deg_part)
  S_pk = _sc_edge(src_pk, dstp, g_pk.reshape(NC * NPAD, H), zeros_ns)
  emb, logits = _tc_final(S_pk, g_pk, deg_part, W1, b1, W2, b2)
  return emb[:N], logits[:N]
